# baseline (speedup 1.0000x reference)
import functools

import jax
import jax.numpy as jnp
from jax import lax
from jax.experimental import pallas as pl
from jax.experimental.pallas import tpu as pltpu
from jax.experimental.pallas import tpu_sc as plsc

NC = 2
NS = 16
L = 16
DH = 128
N_PAD = 10240
RPT = N_PAD // NS
BN = 1024


def _make_deg_kernel(nch):
    nchw = nch // NC
    mesh = plsc.VectorSubcoreMesh(core_axis_name="c", subcore_axis_name="s")

    @functools.partial(
        pl.kernel,
        out_type=jax.ShapeDtypeStruct((NC, N_PAD), jnp.float32),
        mesh=mesh,
        scratch_types=[
            pltpu.VMEM((nchw, 128), jnp.int32),
            pltpu.VMEM((nchw, 128), jnp.float32),
            pltpu.VMEM_SHARED((N_PAD,), jnp.float32),
        ],
        compiler_params=pltpu.CompilerParams(needs_layout_passes=False),
    )
    def deg_kernel(col_hbm, ew_hbm, z_hbm, deg_out, col_v, ew_v, deg_sh):
        c = lax.axis_index("c")
        s = lax.axis_index("s")
        pltpu.sync_copy(z_hbm.at[pl.ds(s * RPT, RPT)],
                        deg_sh.at[pl.ds(s * RPT, RPT)])
        pltpu.sync_copy(col_hbm.at[s, pl.ds(c * nchw, nchw)], col_v)
        pltpu.sync_copy(ew_hbm.at[s, pl.ds(c * nchw, nchw)], ew_v)
        plsc.subcore_barrier()

        def body(j, carry):
            pltpu.sync_copy(ew_v.at[j], deg_sh.at[col_v.at[j]], add=True)
            return carry

        lax.fori_loop(0, nchw, body, 0)
        plsc.subcore_barrier()
        pltpu.sync_copy(deg_sh.at[pl.ds(s * RPT, RPT)],
                        deg_out.at[c, pl.ds(s * RPT, RPT)])

    return deg_kernel


CH = 80
NB = 4
EB = 8


def _make_agg_kernel(nch):
    mesh = plsc.VectorSubcoreMesh(core_axis_name="c", subcore_axis_name="s")
    assert nch % EB == 0 and nch >= 2 * EB

    @functools.partial(
        pl.kernel,
        out_type=jax.ShapeDtypeStruct((NC, N_PAD, DH), jnp.float32),
        mesh=mesh,
        scratch_types=[
            [pltpu.VMEM((3, CH), jnp.int32)] * EB,
            [pltpu.VMEM((CH, DH), jnp.float32)] * NB,
            pltpu.VMEM_SHARED((N_PAD, DH), jnp.float32),
            pltpu.SemaphoreType.DMA,
            [pltpu.SemaphoreType.DMA] * NB,
            [pltpu.SemaphoreType.DMA] * 2,
        ],
        compiler_params=pltpu.CompilerParams(needs_layout_passes=False),
    )
    def agg_kernel(y_hbm, edges_hbm, acc_out, ebufs, gbufs, acc_sh,
                   gsem, ssems, esems):
        c = lax.axis_index("c")
        s = lax.axis_index("s")

        def start_edge(j, m):
            pltpu.async_copy(edges_hbm.at[s, j], ebufs[m % EB],
                             esems[m % 2])

        def wait_edge(j, m):
            pltpu.make_async_copy(edges_hbm.at[s, j], ebufs[m % EB],
                                  esems[m % 2]).wait()

        def start_gather(m, buf):
            pltpu.async_copy(y_hbm.at[c].at[ebufs[m % EB].at[0]], buf, gsem)

        def wait_gather(m, buf):
            pltpu.make_async_copy(y_hbm.at[c].at[ebufs[m % EB].at[0]], buf,
                                  gsem).wait()

        def start_scatter(m, buf, sem):
            pltpu.async_copy(buf, acc_sh.at[ebufs[m % EB].at[1]], sem,
                             add=True)

        def wait_scatter(m, buf, sem):
            pltpu.make_async_copy(buf, acc_sh.at[ebufs[m % EB].at[1]],
                                  sem).wait()

        def scale(m, buf):
            two = jnp.full((L,), 2, jnp.int32)
            ebuf = ebufs[m % EB]

            def edge(e, ecarry):
                ev = jnp.full((L,), e, jnp.int32)
                w = plsc.bitcast(plsc.load_gather(ebuf, [two, ev]),
                                 jnp.float32)
                for g in range(DH // L):
                    buf[e, pl.ds(g * L, L)] = buf[e, pl.ds(g * L, L)] * w
                return ecarry

            lax.fori_loop(0, CH, edge, 0, unroll=4)

        pltpu.async_copy(y_hbm.at[c, pl.ds(s * RPT, RPT)],
                         acc_sh.at[pl.ds(s * RPT, RPT)], gsem)
        start_edge(0, 0)
        start_edge(1, 1)
        pltpu.make_async_copy(y_hbm.at[c, pl.ds(s * RPT, RPT)],
                              acc_sh.at[pl.ds(s * RPT, RPT)], gsem).wait()
        plsc.subcore_barrier()
        wait_edge(0, 0)
        start_gather(0, gbufs[0])

        def super_iter(i, carry):
            for m in range(EB):
                j = EB * i + m
                k = m % NB
                kn = (m + 1) % NB
                wait_gather(m, gbufs[k])
                @pl.when(j >= NB - 1)
                def _():
                    wait_scatter(m + 1, gbufs[kn], ssems[kn])

                @pl.when(j + 2 < nch)
                def _():
                    start_edge(j + 2, m + 2)

                @pl.when(j + 1 < nch)
                def _():
                    wait_edge(j + 1, m + 1)
                    start_gather(m + 1, gbufs[kn])

                scale(m, gbufs[k])
                start_scatter(m, gbufs[k], ssems[k])
            return carry

        lax.fori_loop(0, nch // EB, super_iter, 0)
        for j in range(nch - (NB - 1), nch):
            wait_scatter(j, gbufs[j % NB], ssems[j % NB])
        plsc.subcore_barrier()
        pltpu.sync_copy(acc_sh.at[pl.ds(s * RPT, RPT)],
                        acc_out.at[c, pl.ds(s * RPT, RPT)])

    return agg_kernel


def _dinv_body(deg_ref, out_ref):
    d = deg_ref[0] + deg_ref[1] + 1.0
    out_ref[...] = jnp.where(d > 0.0, lax.rsqrt(d), 0.0)


def _dinv_call(degp):
    return pl.pallas_call(
        _dinv_body,
        grid=(N_PAD // BN,),
        in_specs=[pl.BlockSpec((NC, BN, 1), lambda i: (0, i, 0))],
        out_specs=pl.BlockSpec((BN, 1), lambda i: (i, 0)),
        out_shape=jax.ShapeDtypeStruct((N_PAD, 1), jnp.float32),
    )(degp)


def _mm1_body(x_ref, w_ref, dinv_ref, y_ref):
    xw = lax.dot_general(x_ref[...], w_ref[...],
                         (((1,), (1,)), ((), ())),
                         preferred_element_type=jnp.float32)
    y = xw * dinv_ref[...]
    y_ref[0] = y[:, :DH]
    y_ref[1] = y[:, DH:]


def _mm1_call(x_pad, W1, dinv):
    d = 2 * DH
    return pl.pallas_call(
        _mm1_body,
        grid=(N_PAD // BN,),
        in_specs=[
            pl.BlockSpec((BN, d), lambda i: (i, 0)),
            pl.BlockSpec((d, d), lambda i: (0, 0)),
            pl.BlockSpec((BN, 1), lambda i: (i, 0)),
        ],
        out_specs=pl.BlockSpec((NC, BN, DH), lambda i: (0, i, 0)),
        out_shape=jax.ShapeDtypeStruct((NC, N_PAD, DH), jnp.float32),
    )(x_pad, W1, dinv)


def _mid_body(acc_ref, dinv_ref, b_ref, w_ref, y_ref):
    h = jnp.concatenate([acc_ref[0], acc_ref[1]], axis=1)
    h = jnp.maximum(h * dinv_ref[...] + b_ref[...], 0.0)
    hw = lax.dot_general(h, w_ref[...],
                         (((1,), (1,)), ((), ())),
                         preferred_element_type=jnp.float32)
    y = hw * dinv_ref[...]
    y_ref[0] = y[:, :DH]
    y_ref[1] = y[:, DH:]


def _mid_call(acc1, dinv, b1, W2):
    d = 2 * DH
    return pl.pallas_call(
        _mid_body,
        grid=(N_PAD // BN,),
        in_specs=[
            pl.BlockSpec((NC, BN, DH), lambda i: (0, i, 0)),
            pl.BlockSpec((BN, 1), lambda i: (i, 0)),
            pl.BlockSpec((1, d), lambda i: (0, 0)),
            pl.BlockSpec((d, d), lambda i: (0, 0)),
        ],
        out_specs=pl.BlockSpec((NC, BN, DH), lambda i: (0, i, 0)),
        out_shape=jax.ShapeDtypeStruct((NC, N_PAD, DH), jnp.float32),
    )(acc1, dinv, b1, W2)


def _out_body(acc_ref, dinv_ref, b_ref, o_ref):
    h = jnp.concatenate([acc_ref[0], acc_ref[1]], axis=1)
    o_ref[...] = jnp.maximum(h * dinv_ref[...] + b_ref[...], 0.0)


def _out_call(acc2, dinv, b2):
    d = 2 * DH
    return pl.pallas_call(
        _out_body,
        grid=(N_PAD // BN,),
        in_specs=[
            pl.BlockSpec((NC, BN, DH), lambda i: (0, i, 0)),
            pl.BlockSpec((BN, 1), lambda i: (i, 0)),
            pl.BlockSpec((1, d), lambda i: (0, 0)),
        ],
        out_specs=pl.BlockSpec((BN, d), lambda i: (i, 0)),
        out_shape=jax.ShapeDtypeStruct((N_PAD, d), jnp.float32),
    )(acc2, dinv, b2)


def kernel(x, edge_index, edge_weight, W1, b1, W2, b2):
    N = x.shape[0]
    E = edge_weight.shape[0]
    row = edge_index[0].astype(jnp.int32)
    col = edge_index[1].astype(jnp.int32)
    ew = edge_weight.astype(jnp.float32)
    perm = jnp.argsort(col)
    row, col, ew = row[perm], col[perm], ew[perm]

    nch_d = -(-E // (NS * 128 * NC)) * NC
    pad_d = NS * 128 * nch_d - E
    col3 = jnp.concatenate([col, jnp.full((pad_d,), N, jnp.int32)]).reshape(
        NS, nch_d, 128)
    ew3 = jnp.concatenate([ew, jnp.zeros((pad_d,), jnp.float32)]).reshape(
        NS, nch_d, 128)

    nch = -(-E // (NS * CH * EB)) * EB
    pad = NS * CH * nch - E
    rowp = jnp.concatenate([row, jnp.zeros((pad,), jnp.int32)]).reshape(
        NS, nch, 1, CH)
    colp = jnp.concatenate([col, jnp.full((pad,), N, jnp.int32)]).reshape(
        NS, nch, 1, CH)
    ewp = jax.lax.bitcast_convert_type(
        jnp.concatenate([ew, jnp.zeros((pad,), jnp.float32)]),
        jnp.int32).reshape(NS, nch, 1, CH)
    edges = jnp.concatenate([rowp, colp, ewp], axis=2)

    x_pad = jnp.pad(x, ((0, N_PAD - N), (0, 0)))
    zeros = jnp.zeros((N_PAD,), jnp.float32)

    degp = _make_deg_kernel(nch_d)(col3, ew3, zeros)
    dinv = _dinv_call(degp.reshape(NC, N_PAD, 1))
    agg = _make_agg_kernel(nch)

    y1 = _mm1_call(x_pad, W1, dinv)
    acc1 = agg(y1, edges)
    y2 = _mid_call(acc1, dinv, b1.reshape(1, -1), W2)
    acc2 = agg(y2, edges)
    out = _out_call(acc2, dinv, b2.reshape(1, -1))
    return out[:N]


# speedup vs baseline: 1.3512x; 1.3512x over previous
import functools

import jax
import jax.numpy as jnp
from jax import lax
from jax.experimental import pallas as pl
from jax.experimental.pallas import tpu as pltpu
from jax.experimental.pallas import tpu_sc as plsc

NC = 2
NS = 16
L = 16
DH = 128
DQ = DH // 2
N_PAD = 10240
RPT = N_PAD // NS
BN = 1024
CH = 80
EB = 8


def _make_deg_kernel(nch):
    nchw = nch // NC
    mesh = plsc.VectorSubcoreMesh(core_axis_name="c", subcore_axis_name="s")

    @functools.partial(
        pl.kernel,
        out_type=jax.ShapeDtypeStruct((NC, N_PAD), jnp.float32),
        mesh=mesh,
        scratch_types=[
            pltpu.VMEM((nchw, 128), jnp.int32),
            pltpu.VMEM((nchw, 128), jnp.float32),
            pltpu.VMEM_SHARED((N_PAD,), jnp.float32),
        ],
        compiler_params=pltpu.CompilerParams(needs_layout_passes=False),
    )
    def deg_kernel(col_hbm, ew_hbm, z_hbm, deg_out, col_v, ew_v, deg_sh):
        c = lax.axis_index("c")
        s = lax.axis_index("s")
        pltpu.sync_copy(z_hbm.at[pl.ds(s * RPT, RPT)],
                        deg_sh.at[pl.ds(s * RPT, RPT)])
        pltpu.sync_copy(col_hbm.at[s, pl.ds(c * nchw, nchw)], col_v)
        pltpu.sync_copy(ew_hbm.at[s, pl.ds(c * nchw, nchw)], ew_v)
        plsc.subcore_barrier()

        def body(j, carry):
            pltpu.sync_copy(ew_v.at[j], deg_sh.at[col_v.at[j]], add=True)
            return carry

        lax.fori_loop(0, nchw, body, 0)
        plsc.subcore_barrier()
        pltpu.sync_copy(deg_sh.at[pl.ds(s * RPT, RPT)],
                        deg_out.at[c, pl.ds(s * RPT, RPT)])

    return deg_kernel


def _make_agg_kernel(nch):
    mesh = plsc.VectorSubcoreMesh(core_axis_name="c", subcore_axis_name="s")
    assert nch % EB == 0 and nch >= 2 * EB

    @functools.partial(
        pl.kernel,
        out_type=jax.ShapeDtypeStruct((NC, N_PAD, DH), jnp.float32),
        mesh=mesh,
        scratch_types=[
            [pltpu.VMEM((3, CH), jnp.int32)] * EB,
            [pltpu.VMEM((CH, DQ), jnp.int32)] * 2,
            [pltpu.VMEM((CH, DH), jnp.float32)] * 2,
            pltpu.VMEM_SHARED((N_PAD, DH), jnp.float32),
            pltpu.SemaphoreType.DMA,
            [pltpu.SemaphoreType.DMA] * 2,
            [pltpu.SemaphoreType.DMA] * 2,
        ],
        compiler_params=pltpu.CompilerParams(needs_layout_passes=False,
                                             use_tc_tiling_on_sc=False),
    )
    def agg_kernel(y_hbm, ypk_hbm, edges_hbm, acc_out, ebufs, raws, gscs,
                   acc_sh, gsem, ssems, esems):
        c = lax.axis_index("c")
        s = lax.axis_index("s")

        def start_edge(j, m):
            pltpu.async_copy(edges_hbm.at[s, j], ebufs[m % EB],
                             esems[m % 2])

        def wait_edge(j, m):
            pltpu.make_async_copy(edges_hbm.at[s, j], ebufs[m % EB],
                                  esems[m % 2]).wait()

        def start_gather(m, buf):
            pltpu.async_copy(ypk_hbm.at[c].at[ebufs[m % EB].at[0]], buf,
                             gsem)

        def wait_gather(m, buf):
            pltpu.make_async_copy(ypk_hbm.at[c].at[ebufs[m % EB].at[0]],
                                  buf, gsem).wait()

        def start_scatter(m, buf, sem):
            pltpu.async_copy(buf, acc_sh.at[ebufs[m % EB].at[1]], sem,
                             add=True)

        def wait_scatter(m, buf, sem):
            pltpu.make_async_copy(buf, acc_sh.at[ebufs[m % EB].at[1]],
                                  sem).wait()

        def scale(m, raw, gsc):
            two = jnp.full((L,), 2, jnp.int32)
            ebuf = ebufs[m % EB]

            def edge(e, ecarry):
                ev = jnp.full((L,), e, jnp.int32)
                w = plsc.bitcast(plsc.load_gather(ebuf, [two, ev]),
                                 jnp.float32)
                for g in range(DQ // L):
                    pair = plsc.bitcast(raw[e, pl.ds(g * L, L)],
                                        jnp.bfloat16)
                    a, b = plsc.unpack(pair,
                                       format=plsc.PackFormat.INTERLEAVED)
                    gsc[e, pl.ds(g * L, L)] = a * w
                    gsc[e, pl.ds(DQ + g * L, L)] = b * w
                return ecarry

            lax.fori_loop(0, CH, edge, 0, unroll=4)

        pltpu.async_copy(y_hbm.at[c, pl.ds(s * RPT, RPT)],
                         acc_sh.at[pl.ds(s * RPT, RPT)], gsem)
        start_edge(0, 0)
        start_edge(1, 1)
        pltpu.make_async_copy(y_hbm.at[c, pl.ds(s * RPT, RPT)],
                              acc_sh.at[pl.ds(s * RPT, RPT)], gsem).wait()
        plsc.subcore_barrier()
        wait_edge(0, 0)
        start_gather(0, raws[0])

        def super_iter(i, carry):
            for m in range(EB):
                j = EB * i + m
                r = m % 2
                wait_gather(m, raws[r])

                @pl.when(j + 2 < nch)
                def _():
                    start_edge(j + 2, m + 2)

                @pl.when(j + 1 < nch)
                def _():
                    wait_edge(j + 1, m + 1)
                    start_gather(m + 1, raws[1 - r])

                @pl.when(j >= 2)
                def _():
                    wait_scatter(m, gscs[r], ssems[r])

                scale(m, raws[r], gscs[r])
                start_scatter(m, gscs[r], ssems[r])
            return carry

        lax.fori_loop(0, nch // EB, super_iter, 0)
        for j in range(nch - 2, nch):
            wait_scatter(j, gscs[j % 2], ssems[j % 2])
        plsc.subcore_barrier()
        pltpu.sync_copy(acc_sh.at[pl.ds(s * RPT, RPT)],
                        acc_out.at[c, pl.ds(s * RPT, RPT)])

    return agg_kernel


def _dinv_body(deg_ref, out_ref):
    d = deg_ref[0] + deg_ref[1] + 1.0
    out_ref[...] = jnp.where(d > 0.0, lax.rsqrt(d), 0.0)


def _dinv_call(degp):
    return pl.pallas_call(
        _dinv_body,
        grid=(N_PAD // BN,),
        in_specs=[pl.BlockSpec((NC, BN, 1), lambda i: (0, i, 0))],
        out_specs=pl.BlockSpec((BN, 1), lambda i: (i, 0)),
        out_shape=jax.ShapeDtypeStruct((N_PAD, 1), jnp.float32),
    )(degp)


def _store_y(y_ref, ypk_ref, y):
    for c in range(NC):
        yc = y[:, c * DH:(c + 1) * DH]
        y_ref[c] = yc
        lo = jax.lax.bitcast_convert_type(
            yc[:, :DQ].astype(jnp.bfloat16), jnp.uint16).astype(jnp.uint32)
        hi = jax.lax.bitcast_convert_type(
            yc[:, DQ:].astype(jnp.bfloat16), jnp.uint16).astype(jnp.uint32)
        ypk_ref[c] = jax.lax.bitcast_convert_type(lo | (hi << 16),
                                                  jnp.int32)


def _mm1_body(x_ref, w_ref, dinv_ref, y_ref, ypk_ref):
    xw = lax.dot_general(x_ref[...], w_ref[...],
                         (((1,), (1,)), ((), ())),
                         preferred_element_type=jnp.float32)
    _store_y(y_ref, ypk_ref, xw * dinv_ref[...])


def _mm1_call(x_pad, W1, dinv):
    d = 2 * DH
    return pl.pallas_call(
        _mm1_body,
        grid=(N_PAD // BN,),
        in_specs=[
            pl.BlockSpec((BN, d), lambda i: (i, 0)),
            pl.BlockSpec((d, d), lambda i: (0, 0)),
            pl.BlockSpec((BN, 1), lambda i: (i, 0)),
        ],
        out_specs=[
            pl.BlockSpec((NC, BN, DH), lambda i: (0, i, 0)),
            pl.BlockSpec((NC, BN, DQ), lambda i: (0, i, 0)),
        ],
        out_shape=[
            jax.ShapeDtypeStruct((NC, N_PAD, DH), jnp.float32),
            jax.ShapeDtypeStruct((NC, N_PAD, DQ), jnp.int32),
        ],
    )(x_pad, W1, dinv)


def _mid_body(acc_ref, dinv_ref, b_ref, w_ref, y_ref, ypk_ref):
    h = jnp.concatenate([acc_ref[0], acc_ref[1]], axis=1)
    h = jnp.maximum(h * dinv_ref[...] + b_ref[...], 0.0)
    hw = lax.dot_general(h, w_ref[...],
                         (((1,), (1,)), ((), ())),
                         preferred_element_type=jnp.float32)
    _store_y(y_ref, ypk_ref, hw * dinv_ref[...])


def _mid_call(acc1, dinv, b1, W2):
    d = 2 * DH
    return pl.pallas_call(
        _mid_body,
        grid=(N_PAD // BN,),
        in_specs=[
            pl.BlockSpec((NC, BN, DH), lambda i: (0, i, 0)),
            pl.BlockSpec((BN, 1), lambda i: (i, 0)),
            pl.BlockSpec((1, d), lambda i: (0, 0)),
            pl.BlockSpec((d, d), lambda i: (0, 0)),
        ],
        out_specs=[
            pl.BlockSpec((NC, BN, DH), lambda i: (0, i, 0)),
            pl.BlockSpec((NC, BN, DQ), lambda i: (0, i, 0)),
        ],
        out_shape=[
            jax.ShapeDtypeStruct((NC, N_PAD, DH), jnp.float32),
            jax.ShapeDtypeStruct((NC, N_PAD, DQ), jnp.int32),
        ],
    )(acc1, dinv, b1, W2)


def _out_body(acc_ref, dinv_ref, b_ref, o_ref):
    h = jnp.concatenate([acc_ref[0], acc_ref[1]], axis=1)
    o_ref[...] = jnp.maximum(h * dinv_ref[...] + b_ref[...], 0.0)


def _out_call(acc2, dinv, b2):
    d = 2 * DH
    return pl.pallas_call(
        _out_body,
        grid=(N_PAD // BN,),
        in_specs=[
            pl.BlockSpec((NC, BN, DH), lambda i: (0, i, 0)),
            pl.BlockSpec((BN, 1), lambda i: (i, 0)),
            pl.BlockSpec((1, d), lambda i: (0, 0)),
        ],
        out_specs=pl.BlockSpec((BN, d), lambda i: (i, 0)),
        out_shape=jax.ShapeDtypeStruct((N_PAD, d), jnp.float32),
    )(acc2, dinv, b2)


def kernel(x, edge_index, edge_weight, W1, b1, W2, b2):
    N = x.shape[0]
    E = edge_weight.shape[0]
    row = edge_index[0].astype(jnp.int32)
    col = edge_index[1].astype(jnp.int32)
    ew = edge_weight.astype(jnp.float32)

    nch_d = -(-E // (NS * 128 * NC)) * NC
    pad_d = NS * 128 * nch_d - E
    col3 = jnp.concatenate([col, jnp.full((pad_d,), N, jnp.int32)]).reshape(
        NS, nch_d, 128)
    ew3 = jnp.concatenate([ew, jnp.zeros((pad_d,), jnp.float32)]).reshape(
        NS, nch_d, 128)

    nch = -(-E // (NS * CH * EB)) * EB
    pad = NS * CH * nch - E
    rowp = jnp.concatenate([row, jnp.zeros((pad,), jnp.int32)]).reshape(
        NS, nch, 1, CH)
    colp = jnp.concatenate([col, jnp.full((pad,), N, jnp.int32)]).reshape(
        NS, nch, 1, CH)
    ewp = jax.lax.bitcast_convert_type(
        jnp.concatenate([ew, jnp.zeros((pad,), jnp.float32)]),
        jnp.int32).reshape(NS, nch, 1, CH)
    edges = jnp.concatenate([rowp, colp, ewp], axis=2)

    x_pad = jnp.pad(x, ((0, N_PAD - N), (0, 0)))
    zeros = jnp.zeros((N_PAD,), jnp.float32)

    degp = _make_deg_kernel(nch_d)(col3, ew3, zeros)
    dinv = _dinv_call(degp.reshape(NC, N_PAD, 1))
    agg = _make_agg_kernel(nch)

    y1, y1pk = _mm1_call(x_pad, W1, dinv)
    acc1 = agg(y1, y1pk, edges)
    y2, y2pk = _mid_call(acc1, dinv, b1.reshape(1, -1), W2)
    acc2 = agg(y2, y2pk, edges)
    out = _out_call(acc2, dinv, b2.reshape(1, -1))
    return out[:N]


# speedup vs baseline: 1.4527x; 1.0751x over previous
import functools

import jax
import jax.numpy as jnp
from jax import lax
from jax.experimental import pallas as pl
from jax.experimental.pallas import tpu as pltpu
from jax.experimental.pallas import tpu_sc as plsc

NC = 2
NS = 16
L = 16
DH = 128
DQ = DH // 2
N_PAD = 10240
RPT = N_PAD // NS
BN = 1024
CH = 80
EB = 8


def _make_deg_kernel(nch):
    nchw = nch // NC
    mesh = plsc.VectorSubcoreMesh(core_axis_name="c", subcore_axis_name="s")

    @functools.partial(
        pl.kernel,
        out_type=jax.ShapeDtypeStruct((NC, N_PAD), jnp.float32),
        mesh=mesh,
        scratch_types=[
            pltpu.VMEM((nchw, 128), jnp.int32),
            pltpu.VMEM((nchw, 128), jnp.float32),
            pltpu.VMEM_SHARED((N_PAD,), jnp.float32),
        ],
        compiler_params=pltpu.CompilerParams(needs_layout_passes=False),
    )
    def deg_kernel(col_hbm, ew_hbm, z_hbm, deg_out, col_v, ew_v, deg_sh):
        c = lax.axis_index("c")
        s = lax.axis_index("s")
        pltpu.sync_copy(z_hbm.at[pl.ds(s * RPT, RPT)],
                        deg_sh.at[pl.ds(s * RPT, RPT)])
        pltpu.sync_copy(col_hbm.at[s, pl.ds(c * nchw, nchw)], col_v)
        pltpu.sync_copy(ew_hbm.at[s, pl.ds(c * nchw, nchw)], ew_v)
        plsc.subcore_barrier()

        def body(j, carry):
            pltpu.sync_copy(ew_v.at[j], deg_sh.at[col_v.at[j]], add=True)
            return carry

        lax.fori_loop(0, nchw, body, 0)
        plsc.subcore_barrier()
        pltpu.sync_copy(deg_sh.at[pl.ds(s * RPT, RPT)],
                        deg_out.at[c, pl.ds(s * RPT, RPT)])

    return deg_kernel


def _make_agg_kernel(nch):
    mesh = plsc.VectorSubcoreMesh(core_axis_name="c", subcore_axis_name="s")
    assert nch % EB == 0 and nch >= 2 * EB

    @functools.partial(
        pl.kernel,
        out_type=jax.ShapeDtypeStruct((NC, N_PAD, DH), jnp.float32),
        mesh=mesh,
        scratch_types=[
            [pltpu.VMEM((3, CH), jnp.int32)] * EB,
            [pltpu.VMEM((CH, DQ), jnp.int32)] * 4,
            [pltpu.VMEM((CH, DH), jnp.float32)] * 2,
            pltpu.VMEM_SHARED((N_PAD, DH), jnp.float32),
            pltpu.SemaphoreType.DMA,
            [pltpu.SemaphoreType.DMA] * 2,
            [pltpu.SemaphoreType.DMA] * 4,
        ],
        compiler_params=pltpu.CompilerParams(needs_layout_passes=False,
                                             use_tc_tiling_on_sc=False),
    )
    def agg_kernel(y_hbm, ypk_hbm, edges_hbm, acc_out, ebufs, raws, gscs,
                   acc_sh, gsem, ssems, esems):
        c = lax.axis_index("c")
        s = lax.axis_index("s")

        def start_edge(j, m):
            pltpu.async_copy(edges_hbm.at[s, j], ebufs[m % EB],
                             esems[m % 4])

        def wait_edge(j, m):
            pltpu.make_async_copy(edges_hbm.at[s, j], ebufs[m % EB],
                                  esems[m % 4]).wait()

        def start_gather(m, buf):
            pltpu.async_copy(ypk_hbm.at[c].at[ebufs[m % EB].at[0]], buf,
                             gsem)

        def wait_gather(m, buf):
            pltpu.make_async_copy(ypk_hbm.at[c].at[ebufs[m % EB].at[0]],
                                  buf, gsem).wait()

        def start_scatter(m, buf, sem):
            pltpu.async_copy(buf, acc_sh.at[ebufs[m % EB].at[1]], sem,
                             add=True)

        def wait_scatter(m, buf, sem):
            pltpu.make_async_copy(buf, acc_sh.at[ebufs[m % EB].at[1]],
                                  sem).wait()

        def scale(m, raw, gsc):
            two = jnp.full((L,), 2, jnp.int32)
            ebuf = ebufs[m % EB]

            def edge(e, ecarry):
                ev = jnp.full((L,), e, jnp.int32)
                w = plsc.bitcast(plsc.load_gather(ebuf, [two, ev]),
                                 jnp.float32)
                for g in range(DQ // L):
                    pair = plsc.bitcast(raw[e, pl.ds(g * L, L)],
                                        jnp.bfloat16)
                    a, b = plsc.unpack(pair,
                                       format=plsc.PackFormat.INTERLEAVED)
                    gsc[e, pl.ds(g * L, L)] = a * w
                    gsc[e, pl.ds(DQ + g * L, L)] = b * w
                return ecarry

            lax.fori_loop(0, CH, edge, 0, unroll=4)

        pltpu.async_copy(y_hbm.at[c, pl.ds(s * RPT, RPT)],
                         acc_sh.at[pl.ds(s * RPT, RPT)], gsem)
        for t in range(4):
            start_edge(t, t)
        pltpu.make_async_copy(y_hbm.at[c, pl.ds(s * RPT, RPT)],
                              acc_sh.at[pl.ds(s * RPT, RPT)], gsem).wait()
        plsc.subcore_barrier()
        for t in range(3):
            wait_edge(t, t)
            start_gather(t, raws[t])

        def super_iter(i, carry):
            for m in range(EB):
                j = EB * i + m
                r = m % 4
                g = m % 2
                wait_gather(m, raws[r])

                @pl.when(j + 4 < nch)
                def _():
                    start_edge(j + 4, m + 4)

                @pl.when(j + 3 < nch)
                def _():
                    wait_edge(j + 3, m + 3)
                    start_gather(m + 3, raws[(m + 3) % 4])

                @pl.when(j >= 2)
                def _():
                    wait_scatter(m, gscs[g], ssems[g])

                scale(m, raws[r], gscs[g])
                start_scatter(m, gscs[g], ssems[g])
            return carry

        lax.fori_loop(0, nch // EB, super_iter, 0)
        for j in range(nch - 2, nch):
            wait_scatter(j, gscs[j % 2], ssems[j % 2])
        plsc.subcore_barrier()
        pltpu.sync_copy(acc_sh.at[pl.ds(s * RPT, RPT)],
                        acc_out.at[c, pl.ds(s * RPT, RPT)])

    return agg_kernel


def _dinv_body(deg_ref, out_ref):
    d = deg_ref[0] + deg_ref[1] + 1.0
    out_ref[...] = jnp.where(d > 0.0, lax.rsqrt(d), 0.0)


def _dinv_call(degp):
    return pl.pallas_call(
        _dinv_body,
        grid=(N_PAD // BN,),
        in_specs=[pl.BlockSpec((NC, BN, 1), lambda i: (0, i, 0))],
        out_specs=pl.BlockSpec((BN, 1), lambda i: (i, 0)),
        out_shape=jax.ShapeDtypeStruct((N_PAD, 1), jnp.float32),
    )(degp)


def _store_y(y_ref, ypk_ref, y):
    for c in range(NC):
        yc = y[:, c * DH:(c + 1) * DH]
        y_ref[c] = yc
        lo = jax.lax.bitcast_convert_type(
            yc[:, :DQ].astype(jnp.bfloat16), jnp.uint16).astype(jnp.uint32)
        hi = jax.lax.bitcast_convert_type(
            yc[:, DQ:].astype(jnp.bfloat16), jnp.uint16).astype(jnp.uint32)
        ypk_ref[c] = jax.lax.bitcast_convert_type(lo | (hi << 16),
                                                  jnp.int32)


def _mm1_body(x_ref, w_ref, dinv_ref, y_ref, ypk_ref):
    xw = lax.dot_general(x_ref[...], w_ref[...],
                         (((1,), (1,)), ((), ())),
                         preferred_element_type=jnp.float32)
    _store_y(y_ref, ypk_ref, xw * dinv_ref[...])


def _mm1_call(x_pad, W1, dinv):
    d = 2 * DH
    return pl.pallas_call(
        _mm1_body,
        grid=(N_PAD // BN,),
        in_specs=[
            pl.BlockSpec((BN, d), lambda i: (i, 0)),
            pl.BlockSpec((d, d), lambda i: (0, 0)),
            pl.BlockSpec((BN, 1), lambda i: (i, 0)),
        ],
        out_specs=[
            pl.BlockSpec((NC, BN, DH), lambda i: (0, i, 0)),
            pl.BlockSpec((NC, BN, DQ), lambda i: (0, i, 0)),
        ],
        out_shape=[
            jax.ShapeDtypeStruct((NC, N_PAD, DH), jnp.float32),
            jax.ShapeDtypeStruct((NC, N_PAD, DQ), jnp.int32),
        ],
    )(x_pad, W1, dinv)


def _mid_body(acc_ref, dinv_ref, b_ref, w_ref, y_ref, ypk_ref):
    h = jnp.concatenate([acc_ref[0], acc_ref[1]], axis=1)
    h = jnp.maximum(h * dinv_ref[...] + b_ref[...], 0.0)
    hw = lax.dot_general(h, w_ref[...],
                         (((1,), (1,)), ((), ())),
                         preferred_element_type=jnp.float32)
    _store_y(y_ref, ypk_ref, hw * dinv_ref[...])


def _mid_call(acc1, dinv, b1, W2):
    d = 2 * DH
    return pl.pallas_call(
        _mid_body,
        grid=(N_PAD // BN,),
        in_specs=[
            pl.BlockSpec((NC, BN, DH), lambda i: (0, i, 0)),
            pl.BlockSpec((BN, 1), lambda i: (i, 0)),
            pl.BlockSpec((1, d), lambda i: (0, 0)),
            pl.BlockSpec((d, d), lambda i: (0, 0)),
        ],
        out_specs=[
            pl.BlockSpec((NC, BN, DH), lambda i: (0, i, 0)),
            pl.BlockSpec((NC, BN, DQ), lambda i: (0, i, 0)),
        ],
        out_shape=[
            jax.ShapeDtypeStruct((NC, N_PAD, DH), jnp.float32),
            jax.ShapeDtypeStruct((NC, N_PAD, DQ), jnp.int32),
        ],
    )(acc1, dinv, b1, W2)


def _out_body(acc_ref, dinv_ref, b_ref, o_ref):
    h = jnp.concatenate([acc_ref[0], acc_ref[1]], axis=1)
    o_ref[...] = jnp.maximum(h * dinv_ref[...] + b_ref[...], 0.0)


def _out_call(acc2, dinv, b2):
    d = 2 * DH
    return pl.pallas_call(
        _out_body,
        grid=(N_PAD // BN,),
        in_specs=[
            pl.BlockSpec((NC, BN, DH), lambda i: (0, i, 0)),
            pl.BlockSpec((BN, 1), lambda i: (i, 0)),
            pl.BlockSpec((1, d), lambda i: (0, 0)),
        ],
        out_specs=pl.BlockSpec((BN, d), lambda i: (i, 0)),
        out_shape=jax.ShapeDtypeStruct((N_PAD, d), jnp.float32),
    )(acc2, dinv, b2)


def kernel(x, edge_index, edge_weight, W1, b1, W2, b2):
    N = x.shape[0]
    E = edge_weight.shape[0]
    row = edge_index[0].astype(jnp.int32)
    col = edge_index[1].astype(jnp.int32)
    ew = edge_weight.astype(jnp.float32)

    nch_d = -(-E // (NS * 128 * NC)) * NC
    pad_d = NS * 128 * nch_d - E
    col3 = jnp.concatenate([col, jnp.full((pad_d,), N, jnp.int32)]).reshape(
        NS, nch_d, 128)
    ew3 = jnp.concatenate([ew, jnp.zeros((pad_d,), jnp.float32)]).reshape(
        NS, nch_d, 128)

    nch = -(-E // (NS * CH * EB)) * EB
    pad = NS * CH * nch - E
    rowp = jnp.concatenate([row, jnp.zeros((pad,), jnp.int32)]).reshape(
        NS, nch, 1, CH)
    colp = jnp.concatenate([col, jnp.full((pad,), N, jnp.int32)]).reshape(
        NS, nch, 1, CH)
    ewp = jax.lax.bitcast_convert_type(
        jnp.concatenate([ew, jnp.zeros((pad,), jnp.float32)]),
        jnp.int32).reshape(NS, nch, 1, CH)
    edges = jnp.concatenate([rowp, colp, ewp], axis=2)

    x_pad = jnp.pad(x, ((0, N_PAD - N), (0, 0)))
    zeros = jnp.zeros((N_PAD,), jnp.float32)

    degp = _make_deg_kernel(nch_d)(col3, ew3, zeros)
    dinv = _dinv_call(degp.reshape(NC, N_PAD, 1))
    agg = _make_agg_kernel(nch)

    y1, y1pk = _mm1_call(x_pad, W1, dinv)
    acc1 = agg(y1, y1pk, edges)
    y2, y2pk = _mid_call(acc1, dinv, b1.reshape(1, -1), W2)
    acc2 = agg(y2, y2pk, edges)
    out = _out_call(acc2, dinv, b2.reshape(1, -1))
    return out[:N]
